# Initial kernel scaffold; baseline (speedup 1.0000x reference)
import jax
import jax.numpy as jnp
from jax.experimental import pallas as pl


def kernel(x, edge_index, W0, a0, W1, a1):
    raise NotImplementedError("write your pallas kernel here")


# speedup vs baseline: 44.5180x; 44.5180x over previous
import functools

import jax
import jax.numpy as jnp
from jax import lax
from jax.experimental import pallas as pl
from jax.experimental.pallas import tpu as pltpu
from jax.experimental.pallas import tpu_sc as plsc

N = 10000
E = 320000
NFEAT = 128
NHID = 64
NHEADS = 4
NCLASS = 40
ALPHA = 0.2

NPAD = 10240
F1 = 80
F2 = 48
K = 400
NB = 25
G = K // 16
CH = NPAD // 16
RB = CH // 2
BN = 1000
_HIGH = jax.lax.Precision.HIGHEST


def _leaky(z):
    return jnp.where(z > 0, z, ALPHA * z)


def _elu(z):
    return jnp.where(z > 0, z, jnp.exp(z) - 1.0)


def _tc_a_body(x_ref, w0_ref, a0_ref, htab_ref, eab_ref):
    xb = x_ref[...]
    rows = []
    als = []
    bes = []
    ones = jnp.ones((BN, 1), jnp.float32)
    zpad = jnp.zeros((BN, F1 - NHID - 1), jnp.float32)
    for h in range(NHEADS):
        hp = jnp.dot(xb, w0_ref[h], preferred_element_type=jnp.float32,
                     precision=_HIGH)
        als.append(jnp.dot(hp, a0_ref[h, :NHID].reshape(NHID, 1),
                           preferred_element_type=jnp.float32, precision=_HIGH))
        bes.append(jnp.dot(hp, a0_ref[h, NHID:].reshape(NHID, 1),
                           preferred_element_type=jnp.float32, precision=_HIGH))
        rows.append(jnp.concatenate([hp, ones, zpad], axis=1))
    htab_ref[...] = jnp.stack(rows, axis=0)
    eab_ref[...] = jnp.concatenate(
        als + bes + [jnp.zeros((BN, 8), jnp.float32)], axis=1)


def _tc_a(x, w0, a0):
    return pl.pallas_call(
        _tc_a_body,
        grid=(N // BN,),
        in_specs=[
            pl.BlockSpec((BN, NFEAT), lambda i: (i, 0)),
            pl.BlockSpec((NHEADS, NFEAT, NHID), lambda i: (0, 0, 0)),
            pl.BlockSpec((NHEADS, 2 * NHID), lambda i: (0, 0)),
        ],
        out_specs=[
            pl.BlockSpec((NHEADS, BN, F1), lambda i: (0, i, 0)),
            pl.BlockSpec((BN, 16), lambda i: (i, 0)),
        ],
        out_shape=[
            jax.ShapeDtypeStruct((NHEADS, N, F1), jnp.float32),
            jax.ShapeDtypeStruct((N, 16), jnp.float32),
        ],
    )(x, w0, a0)


def _make_sc_gat(nheads, F):
    mesh = plsc.VectorSubcoreMesh(core_axis_name="c", subcore_axis_name="s")
    fq = F // 16

    @functools.partial(
        pl.kernel,
        out_type=jax.ShapeDtypeStruct((nheads * 2 * NPAD, F), jnp.float32),
        mesh=mesh,
        compiler_params=pltpu.CompilerParams(
            needs_layout_passes=False, use_tc_tiling_on_sc=False),
        scratch_types=[
            pltpu.VMEM((NPAD,), jnp.float32),
            pltpu.VMEM((NPAD,), jnp.float32),
            pltpu.VMEM((NPAD,), jnp.float32),
            pltpu.VMEM((K,), jnp.int32),
            pltpu.VMEM((K,), jnp.int32),
            pltpu.VMEM((K,), jnp.int32),
            pltpu.VMEM((K,), jnp.float32),
            pltpu.VMEM((K, F), jnp.float32),
            pltpu.VMEM_SHARED((NPAD, F), jnp.float32),
            pltpu.SemaphoreType.DMA,
        ],
    )
    def sc(htab, srcE, dstE, abm, acc_out,
           a_b, b_b, m_b, s_b, d_b, g_b, p_b, rows, accsp, sem):
        c = lax.axis_index("c")
        t = lax.axis_index("s")
        zv = jnp.zeros((16,), jnp.float32)

        for h in range(nheads):
            pltpu.sync_copy(abm.at[3 * h + 0], a_b)
            pltpu.sync_copy(abm.at[3 * h + 1], b_b)
            pltpu.sync_copy(abm.at[3 * h + 2], m_b)

            def zb(i, _):
                for q in range(fq):
                    rows[i, pl.ds(q * 16, 16)] = zv
                return 0
            lax.fori_loop(0, RB, zb, 0)
            pltpu.sync_copy(rows.at[pl.ds(0, RB)], accsp.at[pl.ds(t * CH, RB)])
            pltpu.sync_copy(rows.at[pl.ds(0, RB)],
                            accsp.at[pl.ds(t * CH + RB, RB)])
            plsc.subcore_barrier()

            ebase = (c * 16 + t) * (K * NB)

            def batch(b, _):
                base = ebase + b * K
                pltpu.sync_copy(srcE.at[pl.ds(base, K)], s_b)
                pltpu.sync_copy(dstE.at[pl.ds(base, K)], d_b)

                def grp(g, _):
                    o = g * 16
                    sv = s_b[pl.ds(o, 16)]
                    dv = d_b[pl.ds(o, 16)]
                    av = plsc.load_gather(a_b, [dv])
                    bv = plsc.load_gather(b_b, [sv])
                    mv = plsc.load_gather(m_b, [dv])
                    p_b[pl.ds(o, 16)] = jnp.exp(_leaky(av + bv) - mv)
                    g_b[pl.ds(o, 16)] = sv + (h * N)
                    return 0
                lax.fori_loop(0, G, grp, 0)

                pltpu.async_copy(htab.at[g_b], rows, sem).wait()

                def sg(g, _):
                    o = g * 16
                    pv16 = p_b[pl.ds(o, 16)]
                    for lane in range(16):
                        idx = jnp.full((16,), lane, jnp.int32)
                        pv = pv16.at[idx].get(mode="promise_in_bounds")
                        r = o + lane
                        for q in range(fq):
                            rows[r, pl.ds(q * 16, 16)] = (
                                rows[r, pl.ds(q * 16, 16)] * pv)
                    return 0
                lax.fori_loop(0, G, sg, 0)

                pltpu.sync_copy(rows, accsp.at[d_b], add=True)
                return 0
            lax.fori_loop(0, NB, batch, 0)
            plsc.subcore_barrier()

            out_base = (h * 2 + c) * NPAD + t * CH
            pltpu.sync_copy(accsp.at[pl.ds(t * CH, RB)], rows.at[pl.ds(0, RB)])
            pltpu.sync_copy(rows.at[pl.ds(0, RB)], acc_out.at[pl.ds(out_base, RB)])
            pltpu.sync_copy(accsp.at[pl.ds(t * CH + RB, RB)], rows.at[pl.ds(0, RB)])
            pltpu.sync_copy(rows.at[pl.ds(0, RB)],
                            acc_out.at[pl.ds(out_base + RB, RB)])

    return sc


_sc_l1 = _make_sc_gat(NHEADS, F1)
_sc_l2 = _make_sc_gat(1, F2)


def _tc_b_body(acc_ref, w1_ref, a1_ref, htab2_ref, eab2_ref):
    xs = []
    for h in range(NHEADS):
        d = acc_ref[h, 0] + acc_ref[h, 1]
        o = d[:, :NHID] / (d[:, NHID:NHID + 1] + 1e-16)
        xs.append(_elu(o))
    xcat = jnp.concatenate(xs, axis=1)
    hout = jnp.dot(xcat, w1_ref[...], preferred_element_type=jnp.float32,
                   precision=_HIGH)
    a1v = a1_ref[...]
    al2 = jnp.dot(hout, a1v[:NCLASS].reshape(NCLASS, 1),
                  preferred_element_type=jnp.float32, precision=_HIGH)
    be2 = jnp.dot(hout, a1v[NCLASS:].reshape(NCLASS, 1),
                  preferred_element_type=jnp.float32, precision=_HIGH)
    htab2_ref[...] = jnp.concatenate(
        [hout, jnp.ones((BN, 1), jnp.float32),
         jnp.zeros((BN, F2 - NCLASS - 1), jnp.float32)], axis=1)
    eab2_ref[...] = jnp.concatenate(
        [al2, be2, jnp.zeros((BN, 14), jnp.float32)], axis=1)


def _tc_b(acc1, w1, a1):
    return pl.pallas_call(
        _tc_b_body,
        grid=(N // BN,),
        in_specs=[
            pl.BlockSpec((NHEADS, 2, BN, F1), lambda i: (0, 0, i, 0)),
            pl.BlockSpec((NHEADS * NHID, NCLASS), lambda i: (0, 0)),
            pl.BlockSpec((2 * NCLASS,), lambda i: (0,)),
        ],
        out_specs=[
            pl.BlockSpec((BN, F2), lambda i: (i, 0)),
            pl.BlockSpec((BN, 16), lambda i: (i, 0)),
        ],
        out_shape=[
            jax.ShapeDtypeStruct((N, F2), jnp.float32),
            jax.ShapeDtypeStruct((N, 16), jnp.float32),
        ],
    )(acc1, w1, a1)


def _tc_c_body(acc_ref, out_ref):
    d = acc_ref[0] + acc_ref[1]
    o = d[:, :NCLASS] / (d[:, NCLASS:NCLASS + 1] + 1e-16)
    logits = _elu(o)
    mx = jnp.max(logits, axis=1, keepdims=True)
    ls = logits - mx
    out_ref[...] = ls - jnp.log(jnp.sum(jnp.exp(ls), axis=1, keepdims=True))


def _tc_c(acc2):
    return pl.pallas_call(
        _tc_c_body,
        grid=(N // BN,),
        in_specs=[pl.BlockSpec((2, BN, F2), lambda i: (0, i, 0))],
        out_specs=pl.BlockSpec((BN, NCLASS), lambda i: (i, 0)),
        out_shape=jax.ShapeDtypeStruct((N, NCLASS), jnp.float32),
    )(acc2)


def kernel(x, edge_index, W0, a0, W1, a1):
    src = edge_index[0]
    dst = edge_index[1]

    htab1, eab = _tc_a(x, W0, a0)
    al = eab[:, :NHEADS]
    be = eab[:, NHEADS:2 * NHEADS]
    m = _leaky(al + jnp.max(be, axis=0, keepdims=True))
    abm1 = jnp.stack([al.T, be.T, m.T], axis=1).reshape(3 * NHEADS, N)
    abm1 = jnp.pad(abm1, ((0, 0), (0, NPAD - N)))

    acc1 = _sc_l1(htab1.reshape(NHEADS * N, F1), src, dst, abm1)
    acc1 = acc1.reshape(NHEADS, 2, NPAD, F1)

    htab2, eab2 = _tc_b(acc1, W1, a1)
    al2 = eab2[:, 0]
    be2 = eab2[:, 1]
    m2 = _leaky(al2 + jnp.max(be2))
    abm2 = jnp.pad(jnp.stack([al2, be2, m2], axis=0), ((0, 0), (0, NPAD - N)))

    acc2 = _sc_l2(htab2, src, dst, abm2)
    return _tc_c(acc2.reshape(2, NPAD, F2))
